# baseline (speedup 1.0000x reference)
import functools

import jax
import jax.numpy as jnp
from jax import lax
from jax.experimental import pallas as pl
from jax.experimental.pallas import tpu as pltpu
from jax.experimental.pallas import tpu_sc as plsc

N = 10000
D = 128
E = 320000
NC = 2
NS = 16
NW = NC * NS
NPAD = 10240
EPW = E // NW
CH = 40
NCHUNK = EPW // CH
CPB = 50
NBLK = NCHUNK // CPB
RPT = NPAD // NS
LANES = 16
NBUF = 5

_mesh = plsc.VectorSubcoreMesh(core_axis_name="c", subcore_axis_name="s")


@functools.partial(
    pl.kernel,
    mesh=_mesh,
    out_type=jax.ShapeDtypeStruct((NC, NPAD, D), jnp.float32),
    scratch_types=[
        pltpu.VMEM((CPB, CH), jnp.int32),
        pltpu.VMEM((CPB, CH), jnp.int32),
        pltpu.VMEM((CPB * CH + LANES,), jnp.float32),
        [pltpu.VMEM((CH, D), jnp.float32)] * NBUF,
        pltpu.VMEM((16, D), jnp.float32),
        pltpu.VMEM_SHARED((NPAD, D), jnp.float32),
        pltpu.SemaphoreType.DMA,
        pltpu.SemaphoreType.DMA,
    ],
    compiler_params=pltpu.CompilerParams(
        needs_layout_passes=False, use_tc_tiling_on_sc=False),
)
def _sc_edge_agg(h_hbm, src_hbm, dst_hbm, ew_hbm, out_hbm,
                 src_v, dst_v, ew_v, rows, zblk_v, agg_sh, sem_g, sem_s):
    c = lax.axis_index("c")
    s = lax.axis_index("s")
    w = c * NS + s

    zeros16 = jnp.zeros((LANES,), jnp.float32)
    for r in range(16):
        for g in range(D // LANES):
            zblk_v[r, pl.ds(g * LANES, LANES)] = zeros16

    def zero_body(k, carry):
        pltpu.sync_copy(zblk_v, agg_sh.at[pl.ds(s * RPT + k * 16, 16)])
        return carry
    lax.fori_loop(0, RPT // 16, zero_body, 0)

    plsc.subcore_barrier()

    def start_gather(j, buf):
        return pltpu.async_copy(h_hbm.at[src_v.at[j]], buf, sem_g)

    def wait_gather(buf):
        pltpu.make_async_copy(h_hbm.at[src_v.at[0]], buf, sem_g).wait()

    def start_scatter(j, buf):
        pltpu.async_copy(buf, agg_sh.at[dst_v.at[j]], sem_s, add=True)

    def drain_scatter(buf):
        pltpu.make_async_copy(buf, agg_sh.at[dst_v.at[0]], sem_s).wait()

    def block_body(b, carry):
        pltpu.sync_copy(src_hbm.at[w, b], src_v)
        pltpu.sync_copy(dst_hbm.at[w, b], dst_v)
        pltpu.sync_copy(ew_hbm.at[w, b], ew_v.at[pl.ds(0, CPB * CH)])

        for p in range(NBUF):
            start_gather(p, rows[p])

        def round_body(t, rcarry):
            for bs in range(NBUF):
                j = t * NBUF + bs
                buf = rows[bs]
                wait_gather(buf)

                def row_body(i8, icarry):
                    i0 = i8 * 8
                    ew16 = ew_v[pl.ds(j * CH + i0, LANES)]
                    for r in range(8):
                        splat = ew16.at[jnp.full((LANES,), r, jnp.int32)].get(
                            mode="promise_in_bounds")
                        for g in range(D // LANES):
                            sl = pl.ds(g * LANES, LANES)
                            buf[i0 + r, sl] = buf[i0 + r, sl] * splat
                    return icarry
                lax.fori_loop(0, CH // 8, row_body, 0)

                start_scatter(j, buf)

                nbuf3 = rows[(bs + 3) % NBUF]

                @pl.when(j >= 2)
                def _():
                    drain_scatter(nbuf3)

                @pl.when(jnp.logical_and(j >= 2, j <= CPB - NBUF + 1))
                def _():
                    start_gather(j + 3, nbuf3)
            return rcarry
        lax.fori_loop(0, CPB // NBUF, round_body, 0)

        drain_scatter(rows[0])
        drain_scatter(rows[1])
        return carry
    lax.fori_loop(0, NBLK, block_body, 0)

    plsc.subcore_barrier()

    def out_body(k, carry):
        r0 = s * RPT + k * CH
        pltpu.sync_copy(agg_sh.at[pl.ds(r0, CH)], rows[0])
        pltpu.sync_copy(rows[0], out_hbm.at[c, pl.ds(r0, CH)])
        return carry
    lax.fori_loop(0, RPT // CH, out_body, 0)


def _combine_body(a_ref, h_ref, wr_ref, wro_ref, b_ref, o_ref, *, relu):
    agg = a_ref[0] + a_ref[1]
    out = jnp.dot(agg, wr_ref[...], preferred_element_type=jnp.float32)
    out = out + jnp.dot(h_ref[...], wro_ref[...], preferred_element_type=jnp.float32)
    out = out + b_ref[...]
    if relu:
        out = jnp.maximum(out, 0.0)
    o_ref[...] = out


def _combine(agg2, h, Wr, Wro, br, relu):
    BM = 2000
    grid = (N // BM,)
    return pl.pallas_call(
        functools.partial(_combine_body, relu=relu),
        grid=grid,
        in_specs=[
            pl.BlockSpec((2, BM, D), lambda i: (0, i, 0)),
            pl.BlockSpec((BM, D), lambda i: (i, 0)),
            pl.BlockSpec((D, D), lambda i: (0, 0)),
            pl.BlockSpec((D, D), lambda i: (0, 0)),
            pl.BlockSpec((1, D), lambda i: (0, 0)),
        ],
        out_specs=pl.BlockSpec((BM, D), lambda i: (i, 0)),
        out_shape=jax.ShapeDtypeStruct((N, D), jnp.float32),
    )(agg2, h, Wr, Wro, br.reshape(1, D))


def kernel(x, edge_index, edge_weight,
           W_rel_0, b_rel_0, W_root_0,
           W_rel_1, b_rel_1, W_root_1,
           W_rel_2, b_rel_2, W_root_2):
    src4 = edge_index[0].reshape(NW, NBLK, CPB, CH)
    dst4 = edge_index[1].reshape(NW, NBLK, CPB, CH)
    ew3 = edge_weight.reshape(NW, NBLK, CPB * CH)
    params = [(W_rel_0, b_rel_0, W_root_0),
              (W_rel_1, b_rel_1, W_root_1),
              (W_rel_2, b_rel_2, W_root_2)]
    h = x
    for l, (Wr, br, Wro) in enumerate(params):
        agg2 = _sc_edge_agg(h, src4, dst4, ew3)
        h = _combine(agg2, h, Wr, Wro, br, relu=(l < 2))
    return h


# speedup vs baseline: 2.4734x; 2.4734x over previous
import functools

import jax
import jax.numpy as jnp
from jax import lax
from jax.experimental import pallas as pl
from jax.experimental.pallas import tpu as pltpu
from jax.experimental.pallas import tpu_sc as plsc

N = 10000
D = 128
E = 320000
NC = 2
NS = 16
NW = NC * NS
NPAD = 10240
EPW = E // NW
CH = 40
NCHUNK = EPW // CH
CPB = 50
NBLK = NCHUNK // CPB
RPT = NPAD // NS
LANES = 16
NBUF = 5

_mesh = plsc.VectorSubcoreMesh(core_axis_name="c", subcore_axis_name="s")


@functools.partial(
    pl.kernel,
    mesh=_mesh,
    out_type=jax.ShapeDtypeStruct((NC, NPAD, D), jnp.float32),
    scratch_types=[
        pltpu.VMEM((CPB, CH), jnp.int32),
        pltpu.VMEM((CPB, CH), jnp.int32),
        pltpu.VMEM((CPB * CH + LANES,), jnp.float32),
        [pltpu.VMEM((CH, D), jnp.float32)] * NBUF,
        pltpu.VMEM((16, D), jnp.float32),
        pltpu.VMEM_SHARED((NPAD, D), jnp.float32),
        pltpu.SemaphoreType.DMA,
        pltpu.SemaphoreType.DMA,
    ],
    compiler_params=pltpu.CompilerParams(
        needs_layout_passes=False, use_tc_tiling_on_sc=False),
)
def _sc_edge_agg(h_hbm, src_hbm, dst_hbm, ew_hbm, out_hbm,
                 src_v, dst_v, ew_v, rows, zblk_v, agg_sh, sem_g, sem_s):
    c = lax.axis_index("c")
    s = lax.axis_index("s")
    w = c * NS + s

    zeros16 = jnp.zeros((LANES,), jnp.float32)
    for r in range(16):
        for g in range(D // LANES):
            zblk_v[r, pl.ds(g * LANES, LANES)] = zeros16

    def zero_body(k, carry):
        pltpu.sync_copy(zblk_v, agg_sh.at[pl.ds(s * RPT + k * 16, 16)])
        return carry
    lax.fori_loop(0, RPT // 16, zero_body, 0)

    plsc.subcore_barrier()

    def start_gather(j, buf):
        return pltpu.async_copy(h_hbm.at[src_v.at[j]], buf, sem_g)

    def wait_gather(buf):
        pltpu.make_async_copy(h_hbm.at[src_v.at[0]], buf, sem_g).wait()

    def start_scatter(j, buf):
        pltpu.async_copy(buf, agg_sh.at[dst_v.at[j]], sem_s, add=True)

    def drain_scatter(buf):
        pltpu.make_async_copy(buf, agg_sh.at[dst_v.at[0]], sem_s).wait()

    def block_body(b, carry):
        pltpu.sync_copy(src_hbm.at[w, b], src_v)
        pltpu.sync_copy(dst_hbm.at[w, b], dst_v)
        pltpu.sync_copy(ew_hbm.at[w, b], ew_v.at[pl.ds(0, CPB * CH)])

        for p in range(NBUF):
            start_gather(p, rows[p])

        def round_body(t, rcarry):
            for bs in range(NBUF):
                j = t * NBUF + bs
                buf = rows[bs]
                wait_gather(buf)

                nbuf3 = rows[(bs + 3) % NBUF]

                @pl.when(j >= 2)
                def _():
                    drain_scatter(nbuf3)

                @pl.when(jnp.logical_and(j >= 2, j <= CPB - NBUF + 1))
                def _():
                    start_gather(j + 3, nbuf3)

                def row_body(i2, icarry):
                    for r in range(2):
                        i = i2 * 2 + r
                        splat = plsc.load_gather(
                            ew_v,
                            [jnp.full((LANES,), j * CH + i, jnp.int32)])
                        for g in range(D // LANES):
                            sl = pl.ds(g * LANES, LANES)
                            buf[i, sl] = buf[i, sl] * splat
                    return icarry
                lax.fori_loop(0, CH // 2, row_body, 0)

                start_scatter(j, buf)
            return rcarry
        lax.fori_loop(0, CPB // NBUF, round_body, 0)

        drain_scatter(rows[0])
        drain_scatter(rows[1])
        return carry
    lax.fori_loop(0, NBLK, block_body, 0)

    plsc.subcore_barrier()

    def out_body(k, carry):
        r0 = s * RPT + k * CH
        pltpu.sync_copy(agg_sh.at[pl.ds(r0, CH)], rows[0])
        pltpu.sync_copy(rows[0], out_hbm.at[c, pl.ds(r0, CH)])
        return carry
    lax.fori_loop(0, RPT // CH, out_body, 0)


def _combine_body(a_ref, h_ref, wr_ref, wro_ref, b_ref, o_ref, *, relu):
    agg = a_ref[0] + a_ref[1]
    out = jnp.dot(agg, wr_ref[...], preferred_element_type=jnp.float32)
    out = out + jnp.dot(h_ref[...], wro_ref[...], preferred_element_type=jnp.float32)
    out = out + b_ref[...]
    if relu:
        out = jnp.maximum(out, 0.0)
    o_ref[...] = out


def _combine(agg2, h, Wr, Wro, br, relu):
    BM = 2000
    grid = (N // BM,)
    return pl.pallas_call(
        functools.partial(_combine_body, relu=relu),
        grid=grid,
        in_specs=[
            pl.BlockSpec((2, BM, D), lambda i: (0, i, 0)),
            pl.BlockSpec((BM, D), lambda i: (i, 0)),
            pl.BlockSpec((D, D), lambda i: (0, 0)),
            pl.BlockSpec((D, D), lambda i: (0, 0)),
            pl.BlockSpec((1, D), lambda i: (0, 0)),
        ],
        out_specs=pl.BlockSpec((BM, D), lambda i: (i, 0)),
        out_shape=jax.ShapeDtypeStruct((N, D), jnp.float32),
    )(agg2, h, Wr, Wro, br.reshape(1, D))


def kernel(x, edge_index, edge_weight,
           W_rel_0, b_rel_0, W_root_0,
           W_rel_1, b_rel_1, W_root_1,
           W_rel_2, b_rel_2, W_root_2):
    src4 = edge_index[0].reshape(NW, NBLK, CPB, CH)
    dst4 = edge_index[1].reshape(NW, NBLK, CPB, CH)
    ew3 = edge_weight.reshape(NW, NBLK, CPB * CH)
    params = [(W_rel_0, b_rel_0, W_root_0),
              (W_rel_1, b_rel_1, W_root_1),
              (W_rel_2, b_rel_2, W_root_2)]
    h = x
    for l, (Wr, br, Wro) in enumerate(params):
        agg2 = _sc_edge_agg(h, src4, dst4, ew3)
        h = _combine(agg2, h, Wr, Wro, br, relu=(l < 2))
    return h


# speedup vs baseline: 2.5161x; 1.0173x over previous
import functools

import jax
import jax.numpy as jnp
from jax import lax
from jax.experimental import pallas as pl
from jax.experimental.pallas import tpu as pltpu
from jax.experimental.pallas import tpu_sc as plsc

N = 10000
D = 128
E = 320000
NC = 2
NS = 16
NW = NC * NS
NPAD = 10240
EPW = E // NW
CH = 40
NCHUNK = EPW // CH
CPB = 50
NBLK = NCHUNK // CPB
RPT = NPAD // NS
LANES = 16
NBUF = 5
CHP = 48

_mesh = plsc.VectorSubcoreMesh(core_axis_name="c", subcore_axis_name="s")


@functools.partial(
    pl.kernel,
    mesh=_mesh,
    out_type=jax.ShapeDtypeStruct((NC, NPAD, D), jnp.float32),
    scratch_types=[
        pltpu.VMEM((CPB, CH), jnp.int32),
        pltpu.VMEM((CPB, CH), jnp.int32),
        pltpu.VMEM((CPB * CHP,), jnp.float32),
        [pltpu.VMEM((CH, D), jnp.float32)] * NBUF,
        pltpu.VMEM((16, D), jnp.float32),
        pltpu.VMEM_SHARED((NPAD, D), jnp.float32),
        pltpu.SemaphoreType.DMA,
        pltpu.SemaphoreType.DMA,
    ],
    compiler_params=pltpu.CompilerParams(
        needs_layout_passes=False, use_tc_tiling_on_sc=False),
)
def _sc_edge_agg(h_hbm, src_hbm, dst_hbm, ew_hbm, out_hbm,
                 src_v, dst_v, ew_v, rows, zblk_v, agg_sh, sem_g, sem_s):
    c = lax.axis_index("c")
    s = lax.axis_index("s")
    w = c * NS + s

    zeros16 = jnp.zeros((LANES,), jnp.float32)
    for r in range(16):
        for g in range(D // LANES):
            zblk_v[r, pl.ds(g * LANES, LANES)] = zeros16

    def zero_body(k, carry):
        pltpu.sync_copy(zblk_v, agg_sh.at[pl.ds(s * RPT + k * 16, 16)])
        return carry
    lax.fori_loop(0, RPT // 16, zero_body, 0)

    plsc.subcore_barrier()

    def start_gather(j, buf):
        return pltpu.async_copy(h_hbm.at[src_v.at[j]], buf, sem_g)

    def wait_gather(buf):
        pltpu.make_async_copy(h_hbm.at[src_v.at[0]], buf, sem_g).wait()

    def start_scatter(j, buf):
        pltpu.async_copy(buf, agg_sh.at[dst_v.at[j]], sem_s, add=True)

    def drain_scatter(buf):
        pltpu.make_async_copy(buf, agg_sh.at[dst_v.at[0]], sem_s).wait()

    def block_body(b, carry):
        pltpu.sync_copy(src_hbm.at[w, b], src_v)
        pltpu.sync_copy(dst_hbm.at[w, b], dst_v)
        pltpu.sync_copy(ew_hbm.at[w, b], ew_v)

        for p in range(NBUF):
            start_gather(p, rows[p])

        def round_body(t, rcarry):
            for bs in range(NBUF):
                j = t * NBUF + bs
                buf = rows[bs]
                wait_gather(buf)

                nbuf3 = rows[(bs + 3) % NBUF]

                @pl.when(j >= 2)
                def _():
                    drain_scatter(nbuf3)

                @pl.when(jnp.logical_and(j >= 2, j <= CPB - NBUF + 1))
                def _():
                    start_gather(j + 3, nbuf3)

                for grp in range(3):
                    ew16 = ew_v[pl.ds(j * CHP + grp * LANES, LANES)]
                    nrows = LANES if grp < 2 else CH - 2 * LANES

                    def row_body(i4, icarry, grp=grp, ew16=ew16):
                        for q in range(4):
                            r = i4 * 4 + q
                            splat = ew16.at[
                                jnp.full((LANES,), r, jnp.int32)].get(
                                    mode="promise_in_bounds")
                            i = grp * LANES + r
                            for g in range(D // LANES):
                                sl = pl.ds(g * LANES, LANES)
                                buf[i, sl] = buf[i, sl] * splat
                        return icarry
                    lax.fori_loop(0, nrows // 4, row_body, 0)

                start_scatter(j, buf)
            return rcarry
        lax.fori_loop(0, CPB // NBUF, round_body, 0)

        drain_scatter(rows[0])
        drain_scatter(rows[1])
        return carry
    lax.fori_loop(0, NBLK, block_body, 0)

    plsc.subcore_barrier()

    def out_body(k, carry):
        r0 = s * RPT + k * CH
        pltpu.sync_copy(agg_sh.at[pl.ds(r0, CH)], rows[0])
        pltpu.sync_copy(rows[0], out_hbm.at[c, pl.ds(r0, CH)])
        return carry
    lax.fori_loop(0, RPT // CH, out_body, 0)


def _combine_body(a_ref, h_ref, wr_ref, wro_ref, b_ref, o_ref, *, relu):
    agg = a_ref[0] + a_ref[1]
    out = jnp.dot(agg, wr_ref[...], preferred_element_type=jnp.float32)
    out = out + jnp.dot(h_ref[...], wro_ref[...], preferred_element_type=jnp.float32)
    out = out + b_ref[...]
    if relu:
        out = jnp.maximum(out, 0.0)
    o_ref[...] = out


def _combine(agg2, h, Wr, Wro, br, relu):
    BM = 2000
    grid = (N // BM,)
    return pl.pallas_call(
        functools.partial(_combine_body, relu=relu),
        grid=grid,
        in_specs=[
            pl.BlockSpec((2, BM, D), lambda i: (0, i, 0)),
            pl.BlockSpec((BM, D), lambda i: (i, 0)),
            pl.BlockSpec((D, D), lambda i: (0, 0)),
            pl.BlockSpec((D, D), lambda i: (0, 0)),
            pl.BlockSpec((1, D), lambda i: (0, 0)),
        ],
        out_specs=pl.BlockSpec((BM, D), lambda i: (i, 0)),
        out_shape=jax.ShapeDtypeStruct((N, D), jnp.float32),
    )(agg2, h, Wr, Wro, br.reshape(1, D))


def kernel(x, edge_index, edge_weight,
           W_rel_0, b_rel_0, W_root_0,
           W_rel_1, b_rel_1, W_root_1,
           W_rel_2, b_rel_2, W_root_2):
    src4 = edge_index[0].reshape(NW, NBLK, CPB, CH)
    dst4 = edge_index[1].reshape(NW, NBLK, CPB, CH)
    ew3 = jnp.pad(
        edge_weight.reshape(NW, NBLK, CPB, CH),
        ((0, 0), (0, 0), (0, 0), (0, CHP - CH)),
    ).reshape(NW, NBLK, CPB * CHP)
    params = [(W_rel_0, b_rel_0, W_root_0),
              (W_rel_1, b_rel_1, W_root_1),
              (W_rel_2, b_rel_2, W_root_2)]
    h = x
    for l, (Wr, br, Wro) in enumerate(params):
        agg2 = _sc_edge_agg(h, src4, dst4, ew3)
        h = _combine(agg2, h, Wr, Wro, br, relu=(l < 2))
    return h


# speedup vs baseline: 2.6042x; 1.0350x over previous
import functools

import jax
import jax.numpy as jnp
from jax import lax
from jax.experimental import pallas as pl
from jax.experimental.pallas import tpu as pltpu
from jax.experimental.pallas import tpu_sc as plsc

N = 10000
D = 128
E = 320000
NC = 2
NS = 16
NW = NC * NS
NPAD = 10240
EPW = E // NW
CH = 40
NCHUNK = EPW // CH
CPB = 50
NBLK = NCHUNK // CPB
RPT = NPAD // NS
LANES = 16
NBUF = 5
CHP = 48

_mesh = plsc.VectorSubcoreMesh(core_axis_name="c", subcore_axis_name="s")


@functools.partial(
    pl.kernel,
    mesh=_mesh,
    out_type=jax.ShapeDtypeStruct((NC, NPAD, D), jnp.float32),
    scratch_types=[
        pltpu.VMEM((CPB, CH), jnp.int32),
        pltpu.VMEM((CPB, CH), jnp.int32),
        pltpu.VMEM((CPB * CHP,), jnp.float32),
        [pltpu.VMEM((CH, D), jnp.float32)] * NBUF,
        pltpu.VMEM((16, D), jnp.float32),
        pltpu.VMEM_SHARED((NPAD, D), jnp.float32),
        pltpu.SemaphoreType.DMA,
        pltpu.SemaphoreType.DMA,
    ],
    compiler_params=pltpu.CompilerParams(
        needs_layout_passes=False, use_tc_tiling_on_sc=False),
)
def _sc_edge_agg(h_hbm, src_hbm, dst_hbm, ew_hbm, out_hbm,
                 src_v, dst_v, ew_v, rows, zblk_v, agg_sh, sem_g, sem_s):
    c = lax.axis_index("c")
    s = lax.axis_index("s")
    w = c * NS + s

    zeros16 = jnp.zeros((LANES,), jnp.float32)
    for r in range(16):
        for g in range(D // LANES):
            zblk_v[r, pl.ds(g * LANES, LANES)] = zeros16

    def zero_body(k, carry):
        pltpu.async_copy(zblk_v, agg_sh.at[pl.ds(s * RPT + k * 16, 16)],
                         sem_g)
        return carry
    lax.fori_loop(0, RPT // 16, zero_body, 0)

    def zero_drain(k, carry):
        pltpu.make_async_copy(
            zblk_v, agg_sh.at[pl.ds(s * RPT, 16)], sem_g).wait()
        return carry
    lax.fori_loop(0, RPT // 16, zero_drain, 0)

    plsc.subcore_barrier()

    def start_gather(j, buf):
        return pltpu.async_copy(h_hbm.at[src_v.at[j]], buf, sem_g)

    def wait_gather(buf):
        pltpu.make_async_copy(h_hbm.at[src_v.at[0]], buf, sem_g).wait()

    def start_scatter(j, buf):
        pltpu.async_copy(buf, agg_sh.at[dst_v.at[j]], sem_s, add=True)

    def drain_scatter(buf):
        pltpu.make_async_copy(buf, agg_sh.at[dst_v.at[0]], sem_s).wait()

    def block_body(b, carry):
        pltpu.sync_copy(src_hbm.at[w, b], src_v)
        pltpu.sync_copy(dst_hbm.at[w, b], dst_v)
        pltpu.sync_copy(ew_hbm.at[w, b], ew_v)

        for p in range(NBUF):
            start_gather(p, rows[p])

        def round_body(t, rcarry):
            for bs in range(NBUF):
                j = t * NBUF + bs
                buf = rows[bs]
                wait_gather(buf)

                nbuf3 = rows[(bs + 3) % NBUF]

                @pl.when(j >= 2)
                def _():
                    drain_scatter(nbuf3)

                @pl.when(jnp.logical_and(j >= 2, j <= CPB - NBUF + 1))
                def _():
                    start_gather(j + 3, nbuf3)

                for grp in range(3):
                    ew16 = ew_v[pl.ds(j * CHP + grp * LANES, LANES)]
                    nrows = LANES if grp < 2 else CH - 2 * LANES

                    def row_body(i4, icarry, grp=grp, ew16=ew16):
                        for q in range(4):
                            r = i4 * 4 + q
                            splat = ew16.at[
                                jnp.full((LANES,), r, jnp.int32)].get(
                                    mode="promise_in_bounds")
                            i = grp * LANES + r
                            for g in range(D // LANES):
                                sl = pl.ds(g * LANES, LANES)
                                buf[i, sl] = buf[i, sl] * splat
                        return icarry
                    lax.fori_loop(0, nrows // 4, row_body, 0)

                start_scatter(j, buf)
            return rcarry
        lax.fori_loop(0, CPB // NBUF, round_body, 0)

        drain_scatter(rows[0])
        drain_scatter(rows[1])
        return carry
    lax.fori_loop(0, NBLK, block_body, 0)

    plsc.subcore_barrier()

    def out_drain_one():
        pltpu.make_async_copy(
            rows[0], out_hbm.at[c, pl.ds(s * RPT, CH)], sem_s).wait()

    def out_body(k, carry):
        for p in range(2):
            n = k * 2 + p

            @pl.when(n >= 2)
            def _():
                out_drain_one()

            r0 = s * RPT + n * CH
            pltpu.async_copy(agg_sh.at[pl.ds(r0, CH)], rows[p], sem_g).wait()
            pltpu.async_copy(rows[p], out_hbm.at[c, pl.ds(r0, CH)], sem_s)
        return carry
    lax.fori_loop(0, RPT // CH // 2, out_body, 0)
    out_drain_one()
    out_drain_one()


def _combine_body(a_ref, h_ref, wr_ref, wro_ref, b_ref, o_ref, *, relu):
    agg = a_ref[0] + a_ref[1]
    out = jnp.dot(agg, wr_ref[...], preferred_element_type=jnp.float32)
    out = out + jnp.dot(h_ref[...], wro_ref[...], preferred_element_type=jnp.float32)
    out = out + b_ref[...]
    if relu:
        out = jnp.maximum(out, 0.0)
    o_ref[...] = out


def _combine(agg2, h, Wr, Wro, br, relu):
    BM = 2000
    grid = (N // BM,)
    return pl.pallas_call(
        functools.partial(_combine_body, relu=relu),
        grid=grid,
        in_specs=[
            pl.BlockSpec((2, BM, D), lambda i: (0, i, 0)),
            pl.BlockSpec((BM, D), lambda i: (i, 0)),
            pl.BlockSpec((D, D), lambda i: (0, 0)),
            pl.BlockSpec((D, D), lambda i: (0, 0)),
            pl.BlockSpec((1, D), lambda i: (0, 0)),
        ],
        out_specs=pl.BlockSpec((BM, D), lambda i: (i, 0)),
        out_shape=jax.ShapeDtypeStruct((N, D), jnp.float32),
    )(agg2, h, Wr, Wro, br.reshape(1, D))


def kernel(x, edge_index, edge_weight,
           W_rel_0, b_rel_0, W_root_0,
           W_rel_1, b_rel_1, W_root_1,
           W_rel_2, b_rel_2, W_root_2):
    src4 = edge_index[0].reshape(NW, NBLK, CPB, CH)
    dst4 = edge_index[1].reshape(NW, NBLK, CPB, CH)
    ew3 = jnp.pad(
        edge_weight.reshape(NW, NBLK, CPB, CH),
        ((0, 0), (0, 0), (0, 0), (0, CHP - CH)),
    ).reshape(NW, NBLK, CPB * CHP)
    params = [(W_rel_0, b_rel_0, W_root_0),
              (W_rel_1, b_rel_1, W_root_1),
              (W_rel_2, b_rel_2, W_root_2)]
    h = x
    for l, (Wr, br, Wro) in enumerate(params):
        agg2 = _sc_edge_agg(h, src4, dst4, ew3)
        h = _combine(agg2, h, Wr, Wro, br, relu=(l < 2))
    return h
